# Initial kernel scaffold; baseline (speedup 1.0000x reference)
import functools

import jax
import jax.numpy as jnp
from jax import lax
from jax.experimental import pallas as pl
from jax.experimental.pallas import tpu as pltpu
from jax.experimental.pallas import tpu_sc as plsc

N = 10000
D = 128
E = 320000
NC, NS, L = 2, 16, 16
NW = NC * NS
CH = 128
NPAD = N + 16
RPT = NPAD // NS

NCHUNK = -(-E // (NW * CH))
EPAD = NW * NCHUNK * CH


def _sc_aggregate():
  mesh = plsc.VectorSubcoreMesh(
      core_axis_name="c", subcore_axis_name="s", num_cores=NC, num_subcores=NS)

  @functools.partial(
      pl.kernel,
      out_type=(
          jax.ShapeDtypeStruct((NC, NPAD, D), jnp.float32),
          jax.ShapeDtypeStruct((NC, NPAD, L), jnp.float32),
      ),
      mesh=mesh,
      scratch_types=[
          pltpu.VMEM((NCHUNK, CH), jnp.int32),
          pltpu.VMEM((NCHUNK, CH), jnp.int32),
          pltpu.VMEM((CH, D), jnp.float32),
          pltpu.VMEM((CH, L), jnp.float32),
          pltpu.VMEM_SHARED((NPAD, D), jnp.float32),
          pltpu.VMEM_SHARED((NPAD, L), jnp.float32),
          pltpu.SemaphoreType.DMA,
      ],
  )
  def k(x_hbm, src_hbm, dst_hbm, zd_hbm, zl_hbm, ones_hbm,
        agg_out, cnt_out, src_v, dst_v, rows_v, ones_v, agg_sh, cnt_sh, sem):
    cid = lax.axis_index("c")
    sid = lax.axis_index("s")
    wid = cid * NS + sid

    pltpu.sync_copy(zd_hbm, agg_sh.at[pl.ds(sid * RPT, RPT)])
    pltpu.sync_copy(zl_hbm, cnt_sh.at[pl.ds(sid * RPT, RPT)])
    pltpu.sync_copy(ones_hbm, ones_v)
    pltpu.sync_copy(src_hbm.at[wid], src_v)
    pltpu.sync_copy(dst_hbm.at[wid], dst_v)
    plsc.subcore_barrier()

    def body(j, carry):
      pltpu.async_copy(x_hbm.at[src_v.at[j]], rows_v, sem).wait()
      pltpu.sync_copy(rows_v, agg_sh.at[dst_v.at[j]], add=True)
      pltpu.sync_copy(ones_v, cnt_sh.at[dst_v.at[j]], add=True)
      return carry

    lax.fori_loop(0, NCHUNK, body, 0)
    plsc.subcore_barrier()

    pltpu.sync_copy(agg_sh.at[pl.ds(sid * RPT, RPT)],
                    agg_out.at[cid, pl.ds(sid * RPT, RPT)])
    pltpu.sync_copy(cnt_sh.at[pl.ds(sid * RPT, RPT)],
                    cnt_out.at[cid, pl.ds(sid * RPT, RPT)])

  return k


_sc_agg = _sc_aggregate()

R = 1000


def _tc_dense(relu):
  def body(agg0, agg1, cnt0, cnt1, x, wl, wr, b, out):
    cnt = jnp.maximum(cnt0[:, :1] + cnt1[:, :1], 1.0)
    mean = (agg0[...] + agg1[...]) / cnt
    acc = lax.dot_general(mean, wl[...], (((1,), (1,)), ((), ())),
                          preferred_element_type=jnp.float32)
    acc += lax.dot_general(x[...], wr[...], (((1,), (1,)), ((), ())),
                           preferred_element_type=jnp.float32)
    acc += b[...]
    out[...] = jnp.maximum(acc, 0.0) if relu else acc

  return pl.pallas_call(
      body,
      grid=(N // R,),
      in_specs=[
          pl.BlockSpec((R, D), lambda i: (i, 0)),
          pl.BlockSpec((R, D), lambda i: (i, 0)),
          pl.BlockSpec((R, L), lambda i: (i, 0)),
          pl.BlockSpec((R, L), lambda i: (i, 0)),
          pl.BlockSpec((R, D), lambda i: (i, 0)),
          pl.BlockSpec((D, D), lambda i: (0, 0)),
          pl.BlockSpec((D, D), lambda i: (0, 0)),
          pl.BlockSpec((1, D), lambda i: (0, 0)),
      ],
      out_specs=pl.BlockSpec((R, D), lambda i: (i, 0)),
      out_shape=jax.ShapeDtypeStruct((N, D), jnp.float32),
  )


_tc_relu = _tc_dense(True)
_tc_lin = _tc_dense(False)


def _layer(tc, x, src_r, dst_r, zd, zl, ones, Wl, bl, Wr):
  agg_p, cnt_p = _sc_agg(x, src_r, dst_r, zd, zl, ones)
  return tc(agg_p[0, :N], agg_p[1, :N], cnt_p[0, :N], cnt_p[1, :N],
            x, Wl, Wr, bl.reshape(1, D))


def kernel(x, edge_index, W1l, b1l, W1r, W2l, b2l, W2r):
  src = edge_index[0].astype(jnp.int32)
  dst = edge_index[1].astype(jnp.int32)
  pad = EPAD - E
  src_r = jnp.concatenate([src, jnp.zeros((pad,), jnp.int32)]).reshape(
      NW, NCHUNK, CH)
  dst_r = jnp.concatenate([dst, jnp.full((pad,), N, jnp.int32)]).reshape(
      NW, NCHUNK, CH)
  zd = jnp.zeros((RPT, D), jnp.float32)
  zl = jnp.zeros((RPT, L), jnp.float32)
  ones = jnp.ones((CH, L), jnp.float32)

  h = _layer(_tc_relu, x, src_r, dst_r, zd, zl, ones, W1l, b1l, W1r)
  out = _layer(_tc_lin, h, src_r, dst_r, zd, zl, ones, W2l, b2l, W2r)
  return out


# speedup vs baseline: 5.4571x; 5.4571x over previous
import functools

import jax
import jax.numpy as jnp
from jax import lax
from jax.experimental import pallas as pl
from jax.experimental.pallas import tpu as pltpu
from jax.experimental.pallas import tpu_sc as plsc

N = 10000
D = 128
E = 320000
NC, NS, L = 2, 16, 16
NW = NC * NS
DH = D // NC
CH = 128
NPAD = 10112
RPT = NPAD // NS

NCHUNK = -(-E // (NS * CH))
EPAD = NS * NCHUNK * CH


def _fill2d(ref, nrows, ncols, value):
  vec = jnp.full((16,), value, jnp.float32)

  def body(i, c):
    def inner(k, c2):
      ref[i, pl.ds(k * 16, 16)] = vec
      return c2

    return lax.fori_loop(0, ncols // 16, inner, c)

  lax.fori_loop(0, nrows, body, 0)


def _sc_aggregate():
  mesh = plsc.VectorSubcoreMesh(
      core_axis_name="c", subcore_axis_name="s", num_cores=NC, num_subcores=NS)

  @functools.partial(
      pl.kernel,
      out_type=(
          jax.ShapeDtypeStruct((NC, NPAD, DH), jnp.float32),
          jax.ShapeDtypeStruct((NC, NPAD, L), jnp.float32),
      ),
      mesh=mesh,
      compiler_params=pltpu.CompilerParams(use_tc_tiling_on_sc=False),
      scratch_types=[
          pltpu.VMEM((NCHUNK, CH), jnp.int32),
          pltpu.VMEM((NCHUNK, CH), jnp.int32),
          pltpu.VMEM((CH, DH), jnp.float32),
          pltpu.VMEM((CH, L), jnp.float32),
          pltpu.VMEM((CH, L), jnp.float32),
          pltpu.VMEM_SHARED((NPAD, DH), jnp.float32),
          pltpu.VMEM_SHARED((NPAD, L), jnp.float32),
          pltpu.SemaphoreType.DMA,
      ],
  )
  def k(x_hbm, src_hbm, dst_hbm, agg_out, cnt_out,
        src_v, dst_v, rows_v, ones_v, z16_v, agg_sh, cnt_sh, sem):
    cid = lax.axis_index("c")
    sid = lax.axis_index("s")
    wid = cid * NS + sid
    base = sid * RPT

    _fill2d(rows_v, CH, DH, 0.0)
    _fill2d(ones_v, CH, L, 1.0)
    _fill2d(z16_v, CH, L, 0.0)
    for t in range(RPT // CH):
      pltpu.sync_copy(rows_v, agg_sh.at[pl.ds(base + t * CH, CH)])
      pltpu.sync_copy(z16_v, cnt_sh.at[pl.ds(base + t * CH, CH)])
    rem = RPT % CH
    if rem:
      pltpu.sync_copy(rows_v.at[:rem], agg_sh.at[pl.ds(base + RPT - rem, rem)])
      pltpu.sync_copy(z16_v.at[:rem], cnt_sh.at[pl.ds(base + RPT - rem, rem)])
    pltpu.sync_copy(src_hbm.at[wid], src_v)
    pltpu.sync_copy(dst_hbm.at[sid], dst_v)
    plsc.subcore_barrier()

    def body(j, carry):
      pltpu.async_copy(x_hbm.at[src_v.at[j]], rows_v, sem).wait()
      pltpu.sync_copy(rows_v, agg_sh.at[dst_v.at[j]], add=True)
      pltpu.sync_copy(ones_v, cnt_sh.at[dst_v.at[j]], add=True)
      return carry

    lax.fori_loop(0, NCHUNK, body, 0)
    plsc.subcore_barrier()

    pltpu.sync_copy(agg_sh.at[pl.ds(base, RPT)],
                    agg_out.at[cid, pl.ds(base, RPT)])
    pltpu.sync_copy(cnt_sh.at[pl.ds(base, RPT)],
                    cnt_out.at[cid, pl.ds(base, RPT)])

  return k


_sc_agg = _sc_aggregate()

R = 1000


def _tc_dense(relu):
  def body(agg, cnt, x, wl, wr, b, out):
    c = jnp.maximum(cnt[:, :1], 1.0)
    mean = agg[...] / c
    acc = lax.dot_general(mean, wl[...], (((1,), (1,)), ((), ())),
                          preferred_element_type=jnp.float32)
    acc += lax.dot_general(x[...], wr[...], (((1,), (1,)), ((), ())),
                           preferred_element_type=jnp.float32)
    acc += b[...]
    out[...] = jnp.maximum(acc, 0.0) if relu else acc

  return pl.pallas_call(
      body,
      grid=(N // R,),
      in_specs=[
          pl.BlockSpec((R, D), lambda i: (i, 0)),
          pl.BlockSpec((R, L), lambda i: (i, 0)),
          pl.BlockSpec((R, D), lambda i: (i, 0)),
          pl.BlockSpec((D, D), lambda i: (0, 0)),
          pl.BlockSpec((D, D), lambda i: (0, 0)),
          pl.BlockSpec((1, D), lambda i: (0, 0)),
      ],
      out_specs=pl.BlockSpec((R, D), lambda i: (i, 0)),
      out_shape=jax.ShapeDtypeStruct((N, D), jnp.float32),
  )


_tc_relu = _tc_dense(True)
_tc_lin = _tc_dense(False)


def _layer(tc, x, src2_r, dst_r, Wl, bl, Wr):
  x2 = jnp.concatenate([x[:, :DH], x[:, DH:]], axis=0)
  agg_p, cnt_p = _sc_agg(x2, src2_r, dst_r)
  agg = jnp.concatenate([agg_p[0, :N], agg_p[1, :N]], axis=1)
  return tc(agg, cnt_p[0, :N], x, Wl, Wr, bl.reshape(1, D))


def kernel(x, edge_index, W1l, b1l, W1r, W2l, b2l, W2r):
  src = edge_index[0].astype(jnp.int32)
  dst = edge_index[1].astype(jnp.int32)
  pad = EPAD - E
  srcs = jnp.concatenate([src, jnp.zeros((pad,), jnp.int32)]).reshape(
      NS, NCHUNK, CH)
  dst_r = jnp.concatenate([dst, jnp.full((pad,), N, jnp.int32)]).reshape(
      NS, NCHUNK, CH)
  src2_r = jnp.concatenate([srcs[None], srcs[None] + N]).reshape(
      NW, NCHUNK, CH)

  h = _layer(_tc_relu, x, src2_r, dst_r, W1l, b1l, W1r)
  out = _layer(_tc_lin, h, src2_r, dst_r, W2l, b2l, W2r)
  return out
